# baseline (speedup 1.0000x reference)
import functools

import jax
import jax.numpy as jnp
from jax import lax
from jax.experimental import pallas as pl
from jax.experimental.pallas import tpu as pltpu
from jax.experimental.pallas import tpu_sc as plsc

N = 10000
E = 320000
D = 128

NC = 2
NS = 16

N_PAD = 10240
E_PAD = 327680
K = 80
NBUF = 4
EPW0 = 14400
EPW1 = (E_PAD - NS * EPW0) // NS
CH0 = EPW0 // K
CH1 = EPW1 // K


def _matmul_body(x_ref, wt_ref, b_ref, h_ref):
    h_ref[...] = (
        jnp.dot(x_ref[...], wt_ref[...], preferred_element_type=jnp.float32)
        + b_ref[0:1, :]
    )


def _linear(X, Wt, b2):
    blk = 400
    return pl.pallas_call(
        _matmul_body,
        grid=(N // blk,),
        in_specs=[
            pl.BlockSpec((blk, D), lambda i: (i, 0)),
            pl.BlockSpec((D, D), lambda i: (0, 0)),
            pl.BlockSpec((8, D), lambda i: (0, 0)),
        ],
        out_specs=pl.BlockSpec((blk, D), lambda i: (i, 0)),
        out_shape=jax.ShapeDtypeStruct((N, D), jnp.float32),
    )(X, Wt, b2)


def _spmm_body(h_hbm, rows_hbm, cols_hbm, vals_hbm, out_hbm,
               acc,
               r0, r1, r2, r3, c0, c1, c2, c3, w0, w1, w2, w3,
               v0, v1, v2, v3, i0, i1, i2, i3,
               g0, g1, g2, g3, s0, s1, s2, s3):
    cid = lax.axis_index("c")
    sid = lax.axis_index("s")
    bufs = ((r0, c0, w0, v0, i0, g0, s0), (r1, c1, w1, v1, i1, g1, s1),
            (r2, c2, w2, v2, i2, g2, s2), (r3, c3, w3, v3, i3, g3, s3))

    zero16 = jnp.zeros((16,), jnp.float32)

    def zero_row(j, _):
        for q in range(D // 16):
            r0[j, pl.ds(q * 16, 16)] = zero16
        return 0

    lax.fori_loop(0, K, zero_row, 0)
    slab = sid * (N_PAD // NS)
    for t in range(N_PAD // NS // K):
        pltpu.sync_copy(r0, acc.at[pl.ds(slab + t * K, K)])
    plsc.subcore_barrier()

    base = jnp.where(cid == 0, sid * EPW0, NS * EPW0 + sid * EPW1)
    nquads = jnp.where(cid == 0, CH0 // NBUF, CH1 // NBUF)

    def load_idx(c, buf):
        _, cbuf, wbuf, vbuf, isem, _, _ = buf
        off = base + c * K
        pltpu.async_copy(cols_hbm.at[pl.ds(off, K)], cbuf, isem)
        pltpu.async_copy(vals_hbm.at[pl.ds(off, K)], vbuf, isem)
        pltpu.async_copy(rows_hbm.at[pl.ds(off, K)], wbuf, isem)

    def wait_idx(buf):
        _, cbuf, wbuf, vbuf, isem, _, _ = buf
        pltpu.make_async_copy(cols_hbm.at[pl.ds(0, K)], cbuf, isem).wait()
        pltpu.make_async_copy(vals_hbm.at[pl.ds(0, K)], vbuf, isem).wait()
        pltpu.make_async_copy(rows_hbm.at[pl.ds(0, K)], wbuf, isem).wait()

    def issue_gather(buf):
        rbuf, cbuf, _, _, _, gsem, _ = buf
        pltpu.async_copy(h_hbm.at[cbuf], rbuf, gsem)

    load_idx(0, bufs[0])
    load_idx(1, bufs[1])
    wait_idx(bufs[0])
    issue_gather(bufs[0])

    def quad(p, _):
        for b in range(NBUF):
            c = NBUF * p + b
            cur = bufs[b]
            gb = bufs[(b + 1) % NBUF]
            ib = bufs[(b + 2) % NBUF]
            rbuf, cbuf, wbuf, vbuf, isem, gsem, ssem = cur

            @pl.when(c > 1)
            def _():
                pltpu.make_async_copy(ib[0], acc.at[ib[2]], ib[6]).wait()

            load_idx(c + 2, ib)
            wait_idx(gb)
            issue_gather(gb)
            pltpu.make_async_copy(h_hbm.at[cbuf], rbuf, gsem).wait()

            @functools.partial(plsc.parallel_loop, 0, K // 16, unroll=2)
            def scale_grp(jj):
                ev = vbuf[pl.ds(jj * 16, 16)]
                for e in range(16):
                    s = ev[e]
                    j = jj * 16 + e
                    for q in range(D // 16):
                        sl = pl.ds(q * 16, 16)
                        rbuf[j, sl] = rbuf[j, sl] * s
            pltpu.async_copy(rbuf, acc.at[wbuf], ssem, add=True)
        return 0

    lax.fori_loop(0, nquads, quad, 0)
    pltpu.make_async_copy(r2, acc.at[w2], s2).wait()
    pltpu.make_async_copy(r3, acc.at[w3], s3).wait()
    pltpu.make_async_copy(h_hbm.at[c0], r0, g0).wait()
    wait_idx(bufs[1])
    plsc.subcore_barrier()

    pltpu.sync_copy(acc.at[pl.ds(slab, N_PAD // NS)],
                    out_hbm.at[cid, pl.ds(slab, N_PAD // NS)])


def _spmm(H, rows, cols, vals):
    mesh = plsc.VectorSubcoreMesh(
        core_axis_name="c", subcore_axis_name="s", num_cores=NC,
        num_subcores=NS)
    scratch = [pltpu.VMEM_SHARED((N_PAD, D), jnp.float32)]
    scratch += [pltpu.VMEM((K, D), jnp.float32) for _ in range(NBUF)]
    scratch += [pltpu.VMEM((K,), jnp.int32) for _ in range(2 * NBUF)]
    scratch += [pltpu.VMEM((K,), jnp.float32) for _ in range(NBUF)]
    scratch += [pltpu.SemaphoreType.DMA for _ in range(3 * NBUF)]
    return pl.kernel(
        _spmm_body,
        out_type=jax.ShapeDtypeStruct((NC, N_PAD, D), jnp.float32),
        mesh=mesh,
        scratch_types=scratch,
        compiler_params=pltpu.CompilerParams(needs_layout_passes=False),
    )(H, rows, cols, vals)


def _combine_body(p0_ref, p1_ref, o_ref):
    o_ref[...] = jnp.maximum(p0_ref[0] + p1_ref[0], 0.0)


def _combine(P):
    blk = 320
    return pl.pallas_call(
        _combine_body,
        grid=(N_PAD // blk,),
        in_specs=[
            pl.BlockSpec((1, blk, D), lambda i: (0, i, 0)),
            pl.BlockSpec((1, blk, D), lambda i: (1, i, 0)),
        ],
        out_specs=pl.BlockSpec((blk, D), lambda i: (i, 0)),
        out_shape=jax.ShapeDtypeStruct((N_PAD, D), jnp.float32),
    )(P, P)


def kernel(X, edge_index, edge_vals, W, b):
    Wt = W.T
    b2 = jnp.broadcast_to(b, (8, D))
    H = _linear(X, Wt, b2)

    pad = E_PAD + (NBUF - 1) * K - E
    rows = jnp.concatenate([edge_index[0], jnp.zeros((pad,), jnp.int32)])
    cols = jnp.concatenate([edge_index[1], jnp.zeros((pad,), jnp.int32)])
    vals = jnp.concatenate([edge_vals, jnp.zeros((pad,), jnp.float32)])

    P = _spmm(H, rows, cols, vals)
    out = _combine(P)
    return out[:N]


# speedup vs baseline: 1.0678x; 1.0678x over previous
import functools

import jax
import jax.numpy as jnp
from jax import lax
from jax.experimental import pallas as pl
from jax.experimental.pallas import tpu as pltpu
from jax.experimental.pallas import tpu_sc as plsc

N = 10000
E = 320000
D = 128

NC = 2
NS = 16

N_PAD = 10240
E_PAD = 327680
K = 80
NBUF = 4
EPW0 = 14400
EPW1 = (E_PAD - NS * EPW0) // NS
CH0 = EPW0 // K
CH1 = EPW1 // K


def _matmul_body(x_ref, wt_ref, b_ref, h_ref):
    h_ref[...] = (
        jnp.dot(x_ref[...], wt_ref[...], preferred_element_type=jnp.float32)
        + b_ref[0:1, :]
    )


def _linear(X, Wt, b2):
    blk = 400
    return pl.pallas_call(
        _matmul_body,
        grid=(N // blk,),
        in_specs=[
            pl.BlockSpec((blk, D), lambda i: (i, 0)),
            pl.BlockSpec((D, D), lambda i: (0, 0)),
            pl.BlockSpec((8, D), lambda i: (0, 0)),
        ],
        out_specs=pl.BlockSpec((blk, D), lambda i: (i, 0)),
        out_shape=jax.ShapeDtypeStruct((N, D), jnp.float32),
    )(X, Wt, b2)


def _spmm_body(h_hbm, rows_hbm, cols_hbm, vals_hbm, out_hbm,
               acc,
               r0, r1, r2, r3, c0, c1, c2, c3, w0, w1, w2, w3,
               v0, v1, v2, v3, i0, i1, i2, i3,
               g0, g1, g2, g3, s0, s1, s2, s3):
    cid = lax.axis_index("c")
    sid = lax.axis_index("s")
    bufs = ((r0, c0, w0, v0, i0, g0, s0), (r1, c1, w1, v1, i1, g1, s1),
            (r2, c2, w2, v2, i2, g2, s2), (r3, c3, w3, v3, i3, g3, s3))

    zero16 = jnp.zeros((16,), jnp.float32)

    def zero_row(j, _):
        for q in range(D // 16):
            r0[j, pl.ds(q * 16, 16)] = zero16
        return 0

    lax.fori_loop(0, K, zero_row, 0)
    slab = sid * (N_PAD // NS)
    for t in range(N_PAD // NS // K):
        pltpu.sync_copy(r0, acc.at[pl.ds(slab + t * K, K)])
    plsc.subcore_barrier()

    base = jnp.where(cid == 0, sid * EPW0, NS * EPW0 + sid * EPW1)
    nquads = jnp.where(cid == 0, CH0 // NBUF, CH1 // NBUF)

    def load_idx(c, buf):
        _, cbuf, wbuf, vbuf, isem, _, _ = buf
        off = base + c * K
        pltpu.async_copy(cols_hbm.at[pl.ds(off, K)], cbuf, isem)
        pltpu.async_copy(vals_hbm.at[pl.ds(off, K)], vbuf, isem)
        pltpu.async_copy(rows_hbm.at[pl.ds(off, K)], wbuf, isem)

    def wait_idx(buf):
        _, cbuf, wbuf, vbuf, isem, _, _ = buf
        pltpu.make_async_copy(cols_hbm.at[pl.ds(0, K)], cbuf, isem).wait()
        pltpu.make_async_copy(vals_hbm.at[pl.ds(0, K)], vbuf, isem).wait()
        pltpu.make_async_copy(rows_hbm.at[pl.ds(0, K)], wbuf, isem).wait()

    def issue_gather(buf):
        rbuf, cbuf, _, _, _, gsem, _ = buf
        pltpu.async_copy(h_hbm.at[cbuf], rbuf, gsem)

    load_idx(0, bufs[0])
    load_idx(1, bufs[1])
    wait_idx(bufs[0])
    issue_gather(bufs[0])

    def quad(p, _):
        for b in range(NBUF):
            c = NBUF * p + b
            cur = bufs[b]
            gb = bufs[(b + 1) % NBUF]
            ib = bufs[(b + 2) % NBUF]
            rbuf, cbuf, wbuf, vbuf, isem, gsem, ssem = cur

            @pl.when(c > 1)
            def _():
                pltpu.make_async_copy(ib[0], acc.at[ib[2]], ib[6]).wait()

            load_idx(c + 2, ib)
            wait_idx(gb)
            issue_gather(gb)
            pltpu.make_async_copy(h_hbm.at[cbuf], rbuf, gsem).wait()

            def scale_grp(jj, _):
                ev = vbuf[pl.ds(jj * 16, 16)]
                for e in range(16):
                    s = ev[e]
                    j = jj * 16 + e
                    for q in range(D // 16):
                        sl = pl.ds(q * 16, 16)
                        rbuf[j, sl] = rbuf[j, sl] * s
                return 0

            lax.fori_loop(0, K // 16, scale_grp, 0)
            pltpu.async_copy(rbuf, acc.at[wbuf], ssem, add=True)
        return 0

    lax.fori_loop(0, nquads, quad, 0)
    pltpu.make_async_copy(r2, acc.at[w2], s2).wait()
    pltpu.make_async_copy(r3, acc.at[w3], s3).wait()
    pltpu.make_async_copy(h_hbm.at[c0], r0, g0).wait()
    wait_idx(bufs[1])
    plsc.subcore_barrier()

    pltpu.sync_copy(acc.at[pl.ds(slab, N_PAD // NS)],
                    out_hbm.at[cid, pl.ds(slab, N_PAD // NS)])


def _spmm(H, rows, cols, vals):
    mesh = plsc.VectorSubcoreMesh(
        core_axis_name="c", subcore_axis_name="s", num_cores=NC,
        num_subcores=NS)
    scratch = [pltpu.VMEM_SHARED((N_PAD, D), jnp.float32)]
    scratch += [pltpu.VMEM((K, D), jnp.float32) for _ in range(NBUF)]
    scratch += [pltpu.VMEM((K,), jnp.int32) for _ in range(2 * NBUF)]
    scratch += [pltpu.VMEM((K,), jnp.float32) for _ in range(NBUF)]
    scratch += [pltpu.SemaphoreType.DMA for _ in range(3 * NBUF)]
    return pl.kernel(
        _spmm_body,
        out_type=jax.ShapeDtypeStruct((NC, N_PAD, D), jnp.float32),
        mesh=mesh,
        scratch_types=scratch,
        compiler_params=pltpu.CompilerParams(needs_layout_passes=False),
    )(H, rows, cols, vals)


def _combine_body(p0_ref, p1_ref, o_ref):
    o_ref[...] = jnp.maximum(p0_ref[0] + p1_ref[0], 0.0)


def _combine(P):
    blk = 320
    return pl.pallas_call(
        _combine_body,
        grid=(N_PAD // blk,),
        in_specs=[
            pl.BlockSpec((1, blk, D), lambda i: (0, i, 0)),
            pl.BlockSpec((1, blk, D), lambda i: (1, i, 0)),
        ],
        out_specs=pl.BlockSpec((blk, D), lambda i: (i, 0)),
        out_shape=jax.ShapeDtypeStruct((N_PAD, D), jnp.float32),
    )(P, P)


def kernel(X, edge_index, edge_vals, W, b):
    Wt = W.T
    b2 = jnp.broadcast_to(b, (8, D))
    H = _linear(X, Wt, b2)

    pad = E_PAD + (NBUF - 1) * K - E
    rows = jnp.concatenate([edge_index[0], jnp.zeros((pad,), jnp.int32)])
    cols = jnp.concatenate([edge_index[1], jnp.zeros((pad,), jnp.int32)])
    vals = jnp.concatenate([edge_vals, jnp.zeros((pad,), jnp.float32)])

    P = _spmm(H, rows, cols, vals)
    out = _combine(P)
    return out[:N]
